# Initial kernel scaffold; baseline (speedup 1.0000x reference)
import jax
import jax.numpy as jnp
from jax.experimental import pallas as pl


def kernel(x, edge_index, W_src, W_dst, att, bias, gn_weight, gn_bias, gn_mean_scale):
    raise NotImplementedError("write your pallas kernel here")


# speedup vs baseline: 12.0548x; 12.0548x over previous
import functools

import jax
import jax.numpy as jnp
from jax import lax
from jax.experimental import pallas as pl
from jax.experimental.pallas import tpu as pltpu
from jax.experimental.pallas import tpu_sc as plsc

N = 10000
E = 160000
C = 128
H = 2
SLOPE = 0.2
EPS = 1e-5

NC = 2
NS = 16
L = 16

EPRIME = E + N
K = 64
CPT = 10752
NCH = CPT // K
EP = NS * CPT
NP = 10112
RPT = NP // NS

ROW_BLK = 1000


def _proj_body(x_ref, ws_ref, wd_ref, hs_ref, hd_ref):
    xb = x_ref[...]
    hs_ref[...] = jnp.dot(xb, ws_ref[...], preferred_element_type=jnp.float32)
    hd_ref[...] = jnp.dot(xb, wd_ref[...], preferred_element_type=jnp.float32)


def _project(x, W_src, W_dst):
    nb = N // ROW_BLK
    return pl.pallas_call(
        _proj_body,
        grid=(nb, H),
        in_specs=[
            pl.BlockSpec((ROW_BLK, C), lambda i, j: (i, 0)),
            pl.BlockSpec((C, C), lambda i, j: (0, j)),
            pl.BlockSpec((C, C), lambda i, j: (0, j)),
        ],
        out_specs=[
            pl.BlockSpec((ROW_BLK, C), lambda i, j: (j * nb + i, 0)),
            pl.BlockSpec((ROW_BLK, C), lambda i, j: (j * nb + i, 0)),
        ],
        out_shape=[
            jax.ShapeDtypeStruct((H * N, C), jnp.float32),
            jax.ShapeDtypeStruct((H * N, C), jnp.float32),
        ],
    )(x, W_src, W_dst)


def _passa_body(hs_hbm, hd_hbm, src_hbm, dst_hbm, att_hbm,
                ex_hbm, den_hbm,
                srcv, dstv, gidx, didx, sbuf, dbuf, exbuf, tmpa, denp,
                attq, sem1, sem2):
    c = lax.axis_index("c")
    s = lax.axis_index("s")
    cn = c * N
    lane = lax.iota(jnp.int32, L)

    @pl.loop(0, NP // L)
    def _zrow(r):
        denp[pl.ds(r * L, L)] = jnp.zeros((L,), jnp.float32)

    pltpu.sync_copy(att_hbm.at[pl.ds(c * C, C)], attq)

    base = s * CPT

    @pl.loop(0, NCH)
    def _chunk(j):
        g0 = base + j * K
        pltpu.sync_copy(src_hbm.at[pl.ds(g0, K)], srcv)
        pltpu.sync_copy(dst_hbm.at[pl.ds(g0, K)], dstv)

        @pl.loop(0, K // L)
        def _idx(t):
            gidx[pl.ds(t * L, L)] = srcv[pl.ds(t * L, L)] + cn
            didx[pl.ds(t * L, L)] = dstv[pl.ds(t * L, L)] + cn

        cp1 = pltpu.async_copy(hs_hbm.at[gidx], sbuf, sem1)
        cp2 = pltpu.async_copy(hd_hbm.at[didx], dbuf, sem2)
        cp1.wait()
        cp2.wait()

        @pl.loop(0, K // L)
        def _grp(g):
            logv = jnp.zeros((L,), jnp.float32)
            for e in range(L):
                r = g * L + e
                acc = jnp.zeros((L,), jnp.float32)
                for k in range(C // L):
                    sv = sbuf[r, pl.ds(k * L, L)]
                    dv = dbuf[r, pl.ds(k * L, L)]
                    z = sv + dv
                    lr = jnp.maximum(z, z * SLOPE)
                    acc = acc + lr * attq[pl.ds(k * L, L)]
                for sh in (8, 4, 2, 1):
                    tmpa[...] = acc
                    acc = acc + plsc.load_gather(tmpa, [(lane + sh) & (L - 1)])
                logv = jnp.where(lane == e, acc, logv)
            gid0 = g0 + g * L
            mask = (gid0 + lane) < EPRIME
            exv = jnp.where(mask, jnp.exp(logv), 0.0)
            exbuf[pl.ds(g * L, L)] = exv
            plsc.addupdate_scatter(denp, [dstv[pl.ds(g * L, L)]], exv)

        pltpu.sync_copy(exbuf, ex_hbm.at[pl.ds(c * EP + g0, K)])

    pltpu.sync_copy(denp, den_hbm.at[pl.ds((c * NS + s) * NP, NP)])


def _passa(hs2, hd2, srcp, dstp, attf):
    mesh = plsc.VectorSubcoreMesh(core_axis_name="c", subcore_axis_name="s")
    f = functools.partial(
        pl.kernel,
        out_type=[
            jax.ShapeDtypeStruct((H * EP,), jnp.float32),
            jax.ShapeDtypeStruct((H * NS * NP,), jnp.float32),
        ],
        mesh=mesh,
        compiler_params=pltpu.CompilerParams(needs_layout_passes=False),
        scratch_types=[
            pltpu.VMEM((K,), jnp.int32),
            pltpu.VMEM((K,), jnp.int32),
            pltpu.VMEM((K,), jnp.int32),
            pltpu.VMEM((K,), jnp.int32),
            pltpu.VMEM((K, C), jnp.float32),
            pltpu.VMEM((K, C), jnp.float32),
            pltpu.VMEM((K,), jnp.float32),
            pltpu.VMEM((L,), jnp.float32),
            pltpu.VMEM((NP,), jnp.float32),
            pltpu.VMEM((C,), jnp.float32),
            pltpu.SemaphoreType.DMA,
            pltpu.SemaphoreType.DMA,
        ],
    )(_passa_body)
    return f(hs2, hd2, srcp, dstp, attf)


def _passb_body(hs_hbm, src_hbm, dst_hbm, ex_hbm,
                outp_hbm,
                srcv, dstv, gidx, sbuf, exbuf, msgbuf, zbuf, out_spmem, sem1):
    c = lax.axis_index("c")
    s = lax.axis_index("s")
    cn = c * N

    @pl.loop(0, 8)
    def _zrow(r):
        for k in range(C // L):
            zbuf[r, pl.ds(k * L, L)] = jnp.zeros((L,), jnp.float32)

    @pl.loop(0, RPT // 8)
    def _zcopy(i):
        pltpu.sync_copy(zbuf, out_spmem.at[pl.ds(s * RPT + i * 8, 8)])

    plsc.subcore_barrier()

    base = s * CPT

    @pl.loop(0, NCH)
    def _chunk(j):
        g0 = base + j * K
        pltpu.sync_copy(src_hbm.at[pl.ds(g0, K)], srcv)
        pltpu.sync_copy(dst_hbm.at[pl.ds(g0, K)], dstv)
        pltpu.sync_copy(ex_hbm.at[pl.ds(c * EP + g0, K)], exbuf)

        @pl.loop(0, K // L)
        def _idx(t):
            gidx[pl.ds(t * L, L)] = srcv[pl.ds(t * L, L)] + cn

        pltpu.async_copy(hs_hbm.at[gidx], sbuf, sem1).wait()

        @pl.loop(0, K)
        def _edge(e):
            exe = plsc.load_gather(exbuf, [jnp.full((L,), e, jnp.int32)])
            for k in range(C // L):
                msgbuf[e, pl.ds(k * L, L)] = sbuf[e, pl.ds(k * L, L)] * exe

        pltpu.sync_copy(msgbuf, out_spmem.at[dstv], add=True)

    plsc.subcore_barrier()
    pltpu.sync_copy(out_spmem.at[pl.ds(s * RPT, RPT)],
                    outp_hbm.at[pl.ds(c * NP + s * RPT, RPT)])


def _passb(hs2, srcp, dstp, exf):
    mesh = plsc.VectorSubcoreMesh(core_axis_name="c", subcore_axis_name="s")
    f = functools.partial(
        pl.kernel,
        out_type=[jax.ShapeDtypeStruct((H * NP, C), jnp.float32)],
        mesh=mesh,
        compiler_params=pltpu.CompilerParams(needs_layout_passes=False),
        scratch_types=[
            pltpu.VMEM((K,), jnp.int32),
            pltpu.VMEM((K,), jnp.int32),
            pltpu.VMEM((K,), jnp.int32),
            pltpu.VMEM((K, C), jnp.float32),
            pltpu.VMEM((K,), jnp.float32),
            pltpu.VMEM((K, C), jnp.float32),
            pltpu.VMEM((8, C), jnp.float32),
            pltpu.VMEM_SHARED((NP, C), jnp.float32),
            pltpu.SemaphoreType.DMA,
        ],
    )(_passb_body)
    return f(hs2, srcp, dstp, exf)


def _denred_body(den_ref, out_ref):
    out_ref[0:1, :] = jnp.sum(den_ref[:NS, :], axis=0, keepdims=True)
    out_ref[1:2, :] = jnp.sum(den_ref[NS:, :], axis=0, keepdims=True)


def _denred(den):
    return pl.pallas_call(
        _denred_body,
        out_shape=jax.ShapeDtypeStruct((H, NP), jnp.float32),
    )(den.reshape(H * NS, NP))


def _final_body(outp_ref, d0_ref, d1_ref, bias_ref, gw_ref, gb_ref, gms_ref,
                y_ref):
    p0 = outp_ref[:N, :]
    p1 = outp_ref[NP:NP + N, :]
    y = 0.5 * (p0 / d0_ref[...] + p1 / d1_ref[...]) + bias_ref[...]
    mu = jnp.mean(y, axis=0, keepdims=True)
    cen = y - gms_ref[...] * mu
    var = jnp.mean(cen * cen, axis=0, keepdims=True)
    y_ref[...] = gw_ref[...] * cen * lax.rsqrt(var + EPS) + gb_ref[...]


def _finish(outp, d0col, d1col, bias, gw, gb, gms):
    return pl.pallas_call(
        _final_body,
        out_shape=jax.ShapeDtypeStruct((N, C), jnp.float32),
    )(outp, d0col, d1col, bias.reshape(1, C), gw.reshape(1, C),
      gb.reshape(1, C), gms.reshape(1, C))


def kernel(x, edge_index, W_src, W_dst, att, bias, gn_weight, gn_bias, gn_mean_scale):
    loops = jnp.arange(N, dtype=jnp.int32)
    pad = jnp.zeros((EP - EPRIME,), jnp.int32)
    srcp = jnp.concatenate([edge_index[0].astype(jnp.int32), loops, pad])
    dstp = jnp.concatenate([edge_index[1].astype(jnp.int32), loops, pad])

    hs2, hd2 = _project(x, W_src, W_dst)
    exf, den = _passa(hs2, hd2, srcp, dstp, att.reshape(H * C))
    (outp,) = _passb(hs2, srcp, dstp, exf)
    denr = _denred(den)
    d0col = denr[0, :N].reshape(N, 1)
    d1col = denr[1, :N].reshape(N, 1)
    return _finish(outp, d0col, d1col, bias, gn_weight, gn_bias,
                   gn_mean_scale)


# speedup vs baseline: 19.6743x; 1.6321x over previous
import functools

import jax
import jax.numpy as jnp
from jax import lax
from jax.experimental import pallas as pl
from jax.experimental.pallas import tpu as pltpu
from jax.experimental.pallas import tpu_sc as plsc

N = 10000
E = 160000
C = 128
H = 2
SLOPE = 0.2
EPS = 1e-5

NC = 2
NS = 16
L = 16

EPRIME = E + N
K = 64
CPT = 10752
NCH = CPT // K
EP = NS * CPT
NP = 10112
RPT = NP // NS

ROW_BLK = 1000


def _proj_body(x_ref, ws_ref, wd_ref, hs_ref, hd_ref):
    xb = x_ref[...]
    hs_ref[...] = jnp.dot(xb, ws_ref[...], preferred_element_type=jnp.float32)
    hd_ref[...] = jnp.dot(xb, wd_ref[...], preferred_element_type=jnp.float32)


def _project(x, W_src, W_dst):
    nb = N // ROW_BLK
    return pl.pallas_call(
        _proj_body,
        grid=(nb, H),
        in_specs=[
            pl.BlockSpec((ROW_BLK, C), lambda i, j: (i, 0)),
            pl.BlockSpec((C, C), lambda i, j: (0, j)),
            pl.BlockSpec((C, C), lambda i, j: (0, j)),
        ],
        out_specs=[
            pl.BlockSpec((ROW_BLK, C), lambda i, j: (j * nb + i, 0)),
            pl.BlockSpec((ROW_BLK, C), lambda i, j: (j * nb + i, 0)),
        ],
        out_shape=[
            jax.ShapeDtypeStruct((H * N, C), jnp.float32),
            jax.ShapeDtypeStruct((H * N, C), jnp.float32),
        ],
    )(x, W_src, W_dst)


def _edge_body(hs_hbm, hd_hbm, src_hbm, dst_hbm, att_hbm,
               den_hbm, outp_hbm,
               srcv, dstv, gidx, didx, sbuf, dbuf, msgbuf, tmpa, denp,
               attq, zbuf, out_spmem, sem1, sem2):
    c = lax.axis_index("c")
    s = lax.axis_index("s")
    cn = c * N
    lane = lax.iota(jnp.int32, L)

    @pl.loop(0, NP // L)
    def _zden(r):
        denp[pl.ds(r * L, L)] = jnp.zeros((L,), jnp.float32)

    @pl.loop(0, 8)
    def _zrow(r):
        for k in range(C // L):
            zbuf[r, pl.ds(k * L, L)] = jnp.zeros((L,), jnp.float32)

    @pl.loop(0, RPT // 8)
    def _zcopy(i):
        pltpu.sync_copy(zbuf, out_spmem.at[pl.ds(s * RPT + i * 8, 8)])

    pltpu.sync_copy(att_hbm.at[pl.ds(c * C, C)], attq)
    plsc.subcore_barrier()

    base = s * CPT

    @pl.loop(0, NCH)
    def _chunk(j):
        g0 = base + j * K
        pltpu.sync_copy(src_hbm.at[pl.ds(g0, K)], srcv)
        pltpu.sync_copy(dst_hbm.at[pl.ds(g0, K)], dstv)

        @pl.loop(0, K // L)
        def _idx(t):
            gidx[pl.ds(t * L, L)] = srcv[pl.ds(t * L, L)] + cn
            didx[pl.ds(t * L, L)] = dstv[pl.ds(t * L, L)] + cn

        cp1 = pltpu.async_copy(hs_hbm.at[gidx], sbuf, sem1)
        cp2 = pltpu.async_copy(hd_hbm.at[didx], dbuf, sem2)
        cp1.wait()
        cp2.wait()

        @pl.loop(0, K // L)
        def _grp(g):
            exv = jnp.zeros((L,), jnp.float32)
            gid0 = g0 + g * L
            for e in range(L):
                r = g * L + e
                acc = jnp.zeros((L,), jnp.float32)
                for k in range(C // L):
                    sv = sbuf[r, pl.ds(k * L, L)]
                    dv = dbuf[r, pl.ds(k * L, L)]
                    z = sv + dv
                    lr = jnp.maximum(z, z * SLOPE)
                    acc = acc + lr * attq[pl.ds(k * L, L)]
                for sh in (8, 4, 2, 1):
                    tmpa[...] = acc
                    acc = acc + plsc.load_gather(tmpa, [(lane + sh) & (L - 1)])
                exe = jnp.where(gid0 + e < EPRIME, jnp.exp(acc), 0.0)
                exv = jnp.where(lane == e, exe, exv)
                for k in range(C // L):
                    msgbuf[r, pl.ds(k * L, L)] = sbuf[r, pl.ds(k * L, L)] * exe
            plsc.addupdate_scatter(denp, [dstv[pl.ds(g * L, L)]], exv)

        pltpu.sync_copy(msgbuf, out_spmem.at[dstv], add=True)

    pltpu.sync_copy(denp, den_hbm.at[pl.ds((c * NS + s) * NP, NP)])
    plsc.subcore_barrier()
    pltpu.sync_copy(out_spmem.at[pl.ds(s * RPT, RPT)],
                    outp_hbm.at[pl.ds(c * NP + s * RPT, RPT)])


def _edge_pass(hs2, hd2, srcp, dstp, attf):
    mesh = plsc.VectorSubcoreMesh(core_axis_name="c", subcore_axis_name="s")
    f = functools.partial(
        pl.kernel,
        out_type=[
            jax.ShapeDtypeStruct((H * NS * NP,), jnp.float32),
            jax.ShapeDtypeStruct((H * NP, C), jnp.float32),
        ],
        mesh=mesh,
        compiler_params=pltpu.CompilerParams(needs_layout_passes=False),
        scratch_types=[
            pltpu.VMEM((K,), jnp.int32),
            pltpu.VMEM((K,), jnp.int32),
            pltpu.VMEM((K,), jnp.int32),
            pltpu.VMEM((K,), jnp.int32),
            pltpu.VMEM((K, C), jnp.float32),
            pltpu.VMEM((K, C), jnp.float32),
            pltpu.VMEM((K, C), jnp.float32),
            pltpu.VMEM((L,), jnp.float32),
            pltpu.VMEM((NP,), jnp.float32),
            pltpu.VMEM((C,), jnp.float32),
            pltpu.VMEM((8, C), jnp.float32),
            pltpu.VMEM_SHARED((NP, C), jnp.float32),
            pltpu.SemaphoreType.DMA,
            pltpu.SemaphoreType.DMA,
        ],
    )(_edge_body)
    return f(hs2, hd2, srcp, dstp, attf)


def _denred_body(den_ref, out_ref):
    out_ref[0:1, :] = jnp.sum(den_ref[:NS, :], axis=0, keepdims=True)
    out_ref[1:2, :] = jnp.sum(den_ref[NS:, :], axis=0, keepdims=True)


def _denred(den):
    return pl.pallas_call(
        _denred_body,
        out_shape=jax.ShapeDtypeStruct((H, NP), jnp.float32),
    )(den.reshape(H * NS, NP))


def _final_body(outp_ref, d0_ref, d1_ref, bias_ref, gw_ref, gb_ref, gms_ref,
                y_ref):
    p0 = outp_ref[:N, :]
    p1 = outp_ref[NP:NP + N, :]
    y = 0.5 * (p0 / d0_ref[...] + p1 / d1_ref[...]) + bias_ref[...]
    mu = jnp.mean(y, axis=0, keepdims=True)
    cen = y - gms_ref[...] * mu
    var = jnp.mean(cen * cen, axis=0, keepdims=True)
    y_ref[...] = gw_ref[...] * cen * lax.rsqrt(var + EPS) + gb_ref[...]


def _finish(outp, d0col, d1col, bias, gw, gb, gms):
    return pl.pallas_call(
        _final_body,
        out_shape=jax.ShapeDtypeStruct((N, C), jnp.float32),
    )(outp, d0col, d1col, bias.reshape(1, C), gw.reshape(1, C),
      gb.reshape(1, C), gms.reshape(1, C))


def kernel(x, edge_index, W_src, W_dst, att, bias, gn_weight, gn_bias, gn_mean_scale):
    loops = jnp.arange(N, dtype=jnp.int32)
    pad = jnp.zeros((EP - EPRIME,), jnp.int32)
    srcp = jnp.concatenate([edge_index[0].astype(jnp.int32), loops, pad])
    dstp = jnp.concatenate([edge_index[1].astype(jnp.int32), loops, pad])

    hs2, hd2 = _project(x, W_src, W_dst)
    den, outp = _edge_pass(hs2, hd2, srcp, dstp, att.reshape(H * C))
    denr = _denred(den)
    d0col = denr[0, :N].reshape(N, 1)
    d1col = denr[1, :N].reshape(N, 1)
    return _finish(outp, d0col, d1col, bias, gn_weight, gn_bias,
                   gn_mean_scale)


# speedup vs baseline: 24.4238x; 1.2414x over previous
import functools

import jax
import jax.numpy as jnp
from jax import lax
from jax.experimental import pallas as pl
from jax.experimental.pallas import tpu as pltpu
from jax.experimental.pallas import tpu_sc as plsc

N = 10000
E = 160000
C = 128
H = 2
SLOPE = 0.2
EPS = 1e-5

NC = 2
NS = 16
L = 16

EPRIME = E + N
K = 48
CPT = 10752
NCH = CPT // K
EP = NS * CPT
NP = 10112
RPT = NP // NS

ROW_BLK = 1000


def _proj_body(x_ref, ws_ref, wd_ref, hs_ref, hd_ref):
    xb = x_ref[...]
    hs_ref[...] = jnp.dot(xb, ws_ref[...], preferred_element_type=jnp.float32)
    hd_ref[...] = jnp.dot(xb, wd_ref[...], preferred_element_type=jnp.float32)


def _project(x, W_src, W_dst):
    nb = N // ROW_BLK
    return pl.pallas_call(
        _proj_body,
        grid=(nb, H),
        in_specs=[
            pl.BlockSpec((ROW_BLK, C), lambda i, j: (i, 0)),
            pl.BlockSpec((C, C), lambda i, j: (0, j)),
            pl.BlockSpec((C, C), lambda i, j: (0, j)),
        ],
        out_specs=[
            pl.BlockSpec((ROW_BLK, C), lambda i, j: (j * nb + i, 0)),
            pl.BlockSpec((ROW_BLK, C), lambda i, j: (j * nb + i, 0)),
        ],
        out_shape=[
            jax.ShapeDtypeStruct((H * N, C), jnp.float32),
            jax.ShapeDtypeStruct((H * N, C), jnp.float32),
        ],
    )(x, W_src, W_dst)


def _edge_body(hs_hbm, hd_hbm, src_hbm, dst_hbm, att_hbm,
               den_hbm, outp_hbm,
               srcv0, dstv0, gidx0, didx0, sbuf0, dbuf0, msgbuf0,
               srcv1, dstv1, gidx1, didx1, sbuf1, dbuf1, msgbuf1,
               tmpa, denp, attq, zbuf, out_spmem,
               sems0, semd0, sems1, semd1, semw0, semw1):
    c = lax.axis_index("c")
    s = lax.axis_index("s")
    cn = c * N
    lane = lax.iota(jnp.int32, L)

    @pl.loop(0, NP // L)
    def _zden(r):
        denp[pl.ds(r * L, L)] = jnp.zeros((L,), jnp.float32)

    @pl.loop(0, 8)
    def _zrow(r):
        for k in range(C // L):
            zbuf[r, pl.ds(k * L, L)] = jnp.zeros((L,), jnp.float32)

    @pl.loop(0, RPT // 8)
    def _zcopy(i):
        pltpu.sync_copy(zbuf, out_spmem.at[pl.ds(s * RPT + i * 8, 8)])

    pltpu.sync_copy(att_hbm.at[pl.ds(c * C, C)], attq)
    plsc.subcore_barrier()

    base = s * CPT

    def prefetch(jj, srcv, dstv, gidx, didx, sbuf, dbuf, sems, semd):
        g0 = base + jj * K
        pltpu.sync_copy(src_hbm.at[pl.ds(g0, K)], srcv)
        pltpu.sync_copy(dst_hbm.at[pl.ds(g0, K)], dstv)

        @pl.loop(0, K // L)
        def _idx(t):
            gidx[pl.ds(t * L, L)] = srcv[pl.ds(t * L, L)] + cn
            didx[pl.ds(t * L, L)] = dstv[pl.ds(t * L, L)] + cn

        pltpu.async_copy(hs_hbm.at[gidx], sbuf, sems)
        pltpu.async_copy(hd_hbm.at[didx], dbuf, semd)

    def compute(jj, dstv, gidx, didx, sbuf, dbuf, msgbuf,
                sems, semd, semw, first):
        pltpu.make_async_copy(hs_hbm.at[gidx], sbuf, sems).wait()
        pltpu.make_async_copy(hd_hbm.at[didx], dbuf, semd).wait()
        if not first:
            @pl.when(jj >= 2)
            def _():
                pltpu.make_async_copy(
                    msgbuf, out_spmem.at[dstv], semw).wait()
        g0c = base + jj * K

        @pl.loop(0, K // L)
        def _grp(g):
            exv = jnp.zeros((L,), jnp.float32)
            gid0 = g0c + g * L
            for e in range(L):
                r = g * L + e
                acc = jnp.zeros((L,), jnp.float32)
                for k in range(C // L):
                    sv = sbuf[r, pl.ds(k * L, L)]
                    dv = dbuf[r, pl.ds(k * L, L)]
                    z = sv + dv
                    lr = jnp.maximum(z, z * SLOPE)
                    acc = acc + lr * attq[pl.ds(k * L, L)]
                for sh in (8, 4, 2, 1):
                    tmpa[...] = acc
                    acc = acc + plsc.load_gather(tmpa, [(lane + sh) & (L - 1)])
                exe = jnp.where(gid0 + e < EPRIME, jnp.exp(acc), 0.0)
                exv = jnp.where(lane == e, exe, exv)
                for k in range(C // L):
                    msgbuf[r, pl.ds(k * L, L)] = sbuf[r, pl.ds(k * L, L)] * exe
            plsc.addupdate_scatter(denp, [dstv[pl.ds(g * L, L)]], exv)

        pltpu.async_copy(msgbuf, out_spmem.at[dstv], semw, add=True)

    prefetch(0, srcv0, dstv0, gidx0, didx0, sbuf0, dbuf0, sems0, semd0)

    @pl.loop(0, NCH, step=2)
    def _chunk(j):
        prefetch(j + 1, srcv1, dstv1, gidx1, didx1, sbuf1, dbuf1,
                 sems1, semd1)
        compute(j, dstv0, gidx0, didx0, sbuf0, dbuf0, msgbuf0,
                sems0, semd0, semw0, first=False)

        @pl.when(j + 2 < NCH)
        def _():
            prefetch(j + 2, srcv0, dstv0, gidx0, didx0, sbuf0, dbuf0,
                     sems0, semd0)

        compute(j + 1, dstv1, gidx1, didx1, sbuf1, dbuf1, msgbuf1,
                sems1, semd1, semw1, first=False)

    pltpu.make_async_copy(msgbuf0, out_spmem.at[dstv0], semw0).wait()
    pltpu.make_async_copy(msgbuf1, out_spmem.at[dstv1], semw1).wait()

    pltpu.sync_copy(denp, den_hbm.at[pl.ds((c * NS + s) * NP, NP)])
    plsc.subcore_barrier()
    pltpu.sync_copy(out_spmem.at[pl.ds(s * RPT, RPT)],
                    outp_hbm.at[pl.ds(c * NP + s * RPT, RPT)])


def _edge_pass(hs2, hd2, srcp, dstp, attf):
    mesh = plsc.VectorSubcoreMesh(core_axis_name="c", subcore_axis_name="s")
    f = functools.partial(
        pl.kernel,
        out_type=[
            jax.ShapeDtypeStruct((H * NS * NP,), jnp.float32),
            jax.ShapeDtypeStruct((H * NP, C), jnp.float32),
        ],
        mesh=mesh,
        compiler_params=pltpu.CompilerParams(needs_layout_passes=False),
        scratch_types=(
            [pltpu.VMEM((K,), jnp.int32)] * 4
            + [pltpu.VMEM((K, C), jnp.float32)] * 3
            + [pltpu.VMEM((K,), jnp.int32)] * 4
            + [pltpu.VMEM((K, C), jnp.float32)] * 3
            + [
                pltpu.VMEM((L,), jnp.float32),
                pltpu.VMEM((NP,), jnp.float32),
                pltpu.VMEM((C,), jnp.float32),
                pltpu.VMEM((8, C), jnp.float32),
                pltpu.VMEM_SHARED((NP, C), jnp.float32),
            ]
            + [pltpu.SemaphoreType.DMA] * 6
        ),
    )(_edge_body)
    return f(hs2, hd2, srcp, dstp, attf)


def _denred_body(den_ref, out_ref):
    out_ref[0:1, :] = jnp.sum(den_ref[:NS, :], axis=0, keepdims=True)
    out_ref[1:2, :] = jnp.sum(den_ref[NS:, :], axis=0, keepdims=True)


def _denred(den):
    return pl.pallas_call(
        _denred_body,
        out_shape=jax.ShapeDtypeStruct((H, NP), jnp.float32),
    )(den.reshape(H * NS, NP))


def _final_body(outp_ref, d0_ref, d1_ref, bias_ref, gw_ref, gb_ref, gms_ref,
                y_ref):
    p0 = outp_ref[:N, :]
    p1 = outp_ref[NP:NP + N, :]
    y = 0.5 * (p0 / d0_ref[...] + p1 / d1_ref[...]) + bias_ref[...]
    mu = jnp.mean(y, axis=0, keepdims=True)
    cen = y - gms_ref[...] * mu
    var = jnp.mean(cen * cen, axis=0, keepdims=True)
    y_ref[...] = gw_ref[...] * cen * lax.rsqrt(var + EPS) + gb_ref[...]


def _finish(outp, d0col, d1col, bias, gw, gb, gms):
    return pl.pallas_call(
        _final_body,
        out_shape=jax.ShapeDtypeStruct((N, C), jnp.float32),
    )(outp, d0col, d1col, bias.reshape(1, C), gw.reshape(1, C),
      gb.reshape(1, C), gms.reshape(1, C))


def kernel(x, edge_index, W_src, W_dst, att, bias, gn_weight, gn_bias, gn_mean_scale):
    loops = jnp.arange(N, dtype=jnp.int32)
    pad = jnp.zeros((EP - EPRIME,), jnp.int32)
    srcp = jnp.concatenate([edge_index[0].astype(jnp.int32), loops, pad])
    dstp = jnp.concatenate([edge_index[1].astype(jnp.int32), loops, pad])

    hs2, hd2 = _project(x, W_src, W_dst)
    den, outp = _edge_pass(hs2, hd2, srcp, dstp, att.reshape(H * C))
    denr = _denred(den)
    d0col = denr[0, :N].reshape(N, 1)
    d1col = denr[1, :N].reshape(N, 1)
    return _finish(outp, d0col, d1col, bias, gn_weight, gn_bias,
                   gn_mean_scale)


# speedup vs baseline: 28.9832x; 1.1867x over previous
import functools

import numpy as np

import jax
import jax.numpy as jnp
from jax import lax
from jax.experimental import pallas as pl
from jax.experimental.pallas import tpu as pltpu
from jax.experimental.pallas import tpu_sc as plsc

N = 10000
E = 160000
C = 128
H = 2
SLOPE = 0.2
EPS = 1e-5

NC = 2
NS = 16
L = 16

EPRIME = E + N
K = 48
CPT = 10752
NCH = CPT // K
EP = NS * CPT
NP = 10112
RPT = NP // NS

ROW_BLK = 1000


def _proj_body(x_ref, ws_ref, wd_ref, hs_ref, hd_ref):
    xb = x_ref[...]
    hs_ref[...] = jnp.dot(xb, ws_ref[...], preferred_element_type=jnp.float32)
    hd_ref[...] = jnp.dot(xb, wd_ref[...], preferred_element_type=jnp.float32)


def _project(x, W_src, W_dst):
    nb = N // ROW_BLK
    return pl.pallas_call(
        _proj_body,
        grid=(nb, H),
        in_specs=[
            pl.BlockSpec((ROW_BLK, C), lambda i, j: (i, 0)),
            pl.BlockSpec((C, C), lambda i, j: (0, j)),
            pl.BlockSpec((C, C), lambda i, j: (0, j)),
        ],
        out_specs=[
            pl.BlockSpec((ROW_BLK, C), lambda i, j: (j * nb + i, 0)),
            pl.BlockSpec((ROW_BLK, C), lambda i, j: (j * nb + i, 0)),
        ],
        out_shape=[
            jax.ShapeDtypeStruct((H * N, C), jnp.float32),
            jax.ShapeDtypeStruct((H * N, C), jnp.float32),
        ],
    )(x, W_src, W_dst)


def _edge_body(hs_hbm, hd_hbm, src_hbm, dst_hbm, att_hbm,
               den_hbm, outp_hbm,
               srcv0, dstv0, gidx0, didx0, sbuf0, dbuf0, msgbuf0,
               srcv1, dstv1, gidx1, didx1, sbuf1, dbuf1, msgbuf1,
               tmpa, denp, attq, zbuf, out_spmem,
               sems0, semd0, sems1, semd1, semw0, semw1):
    c = lax.axis_index("c")
    s = lax.axis_index("s")
    cn = c * N
    lane = lax.iota(jnp.int32, L)

    @pl.loop(0, NP // L)
    def _zden(r):
        denp[pl.ds(r * L, L)] = jnp.zeros((L,), jnp.float32)

    @pl.loop(0, 8)
    def _zrow(r):
        for k in range(C // L):
            zbuf[r, pl.ds(k * L, L)] = jnp.zeros((L,), jnp.float32)

    @pl.loop(0, RPT // 8)
    def _zcopy(i):
        pltpu.sync_copy(zbuf, out_spmem.at[pl.ds(s * RPT + i * 8, 8)])

    pltpu.sync_copy(att_hbm.at[pl.ds(c * C, C)], attq)
    plsc.subcore_barrier()

    base = s * CPT

    def prefetch(jj, srcv, dstv, gidx, didx, sbuf, dbuf, sems, semd):
        g0 = base + jj * K
        pltpu.sync_copy(src_hbm.at[pl.ds(g0, K)], srcv)
        pltpu.sync_copy(dst_hbm.at[pl.ds(g0, K)], dstv)

        @pl.loop(0, K // L)
        def _idx(t):
            gidx[pl.ds(t * L, L)] = srcv[pl.ds(t * L, L)] + cn
            didx[pl.ds(t * L, L)] = dstv[pl.ds(t * L, L)] + cn

        pltpu.async_copy(hs_hbm.at[gidx], sbuf, sems)
        pltpu.async_copy(hd_hbm.at[didx], dbuf, semd)

    def compute(jj, dstv, gidx, didx, sbuf, dbuf, msgbuf,
                sems, semd, semw, first):
        pltpu.make_async_copy(hs_hbm.at[gidx], sbuf, sems).wait()
        pltpu.make_async_copy(hd_hbm.at[didx], dbuf, semd).wait()
        if not first:
            @pl.when(jj >= 2)
            def _():
                pltpu.make_async_copy(
                    msgbuf, out_spmem.at[dstv], semw).wait()
        g0c = base + jj * K

        @pl.loop(0, K // L)
        def _grp(g):
            exv = jnp.zeros((L,), jnp.float32)
            gid0 = g0c + g * L
            for e in range(L):
                r = g * L + e
                acc = jnp.zeros((L,), jnp.float32)
                for k in range(C // L):
                    sv = sbuf[r, pl.ds(k * L, L)]
                    dv = dbuf[r, pl.ds(k * L, L)]
                    z = sv + dv
                    lr = jnp.maximum(z, z * SLOPE)
                    acc = acc + lr * attq[pl.ds(k * L, L)]
                logit = jnp.full((L,), lax.reduce_sum(acc, (0,)))
                exe = jnp.where(gid0 + e < EPRIME, jnp.exp(logit), 0.0)
                exv = jnp.where(lane == e, exe, exv)
                for k in range(C // L):
                    msgbuf[r, pl.ds(k * L, L)] = sbuf[r, pl.ds(k * L, L)] * exe
            plsc.addupdate_scatter(denp, [dstv[pl.ds(g * L, L)]], exv)

        pltpu.async_copy(msgbuf, out_spmem.at[dstv], semw, add=True)

    prefetch(0, srcv0, dstv0, gidx0, didx0, sbuf0, dbuf0, sems0, semd0)

    @pl.loop(0, NCH, step=2)
    def _chunk(j):
        prefetch(j + 1, srcv1, dstv1, gidx1, didx1, sbuf1, dbuf1,
                 sems1, semd1)
        compute(j, dstv0, gidx0, didx0, sbuf0, dbuf0, msgbuf0,
                sems0, semd0, semw0, first=False)

        @pl.when(j + 2 < NCH)
        def _():
            prefetch(j + 2, srcv0, dstv0, gidx0, didx0, sbuf0, dbuf0,
                     sems0, semd0)

        compute(j + 1, dstv1, gidx1, didx1, sbuf1, dbuf1, msgbuf1,
                sems1, semd1, semw1, first=False)

    pltpu.make_async_copy(msgbuf0, out_spmem.at[dstv0], semw0).wait()
    pltpu.make_async_copy(msgbuf1, out_spmem.at[dstv1], semw1).wait()

    pltpu.sync_copy(denp, den_hbm.at[pl.ds((c * NS + s) * NP, NP)])
    plsc.subcore_barrier()
    pltpu.sync_copy(out_spmem.at[pl.ds(s * RPT, RPT)],
                    outp_hbm.at[pl.ds(c * NP + s * RPT, RPT)])


def _edge_pass(hs2, hd2, srcp, dstp, attf):
    mesh = plsc.VectorSubcoreMesh(core_axis_name="c", subcore_axis_name="s")
    f = functools.partial(
        pl.kernel,
        out_type=[
            jax.ShapeDtypeStruct((H * NS * NP,), jnp.float32),
            jax.ShapeDtypeStruct((H * NP, C), jnp.float32),
        ],
        mesh=mesh,
        compiler_params=pltpu.CompilerParams(needs_layout_passes=False),
        scratch_types=(
            [pltpu.VMEM((K,), jnp.int32)] * 4
            + [pltpu.VMEM((K, C), jnp.float32)] * 3
            + [pltpu.VMEM((K,), jnp.int32)] * 4
            + [pltpu.VMEM((K, C), jnp.float32)] * 3
            + [
                pltpu.VMEM((L,), jnp.float32),
                pltpu.VMEM((NP,), jnp.float32),
                pltpu.VMEM((C,), jnp.float32),
                pltpu.VMEM((8, C), jnp.float32),
                pltpu.VMEM_SHARED((NP, C), jnp.float32),
            ]
            + [pltpu.SemaphoreType.DMA] * 6
        ),
    )(_edge_body)
    return f(hs2, hd2, srcp, dstp, attf)


def _denred_body(den_ref, out_ref):
    out_ref[0:1, :] = jnp.sum(den_ref[:NS, :], axis=0, keepdims=True)
    out_ref[1:2, :] = jnp.sum(den_ref[NS:, :], axis=0, keepdims=True)


def _denred(den):
    return pl.pallas_call(
        _denred_body,
        out_shape=jax.ShapeDtypeStruct((H, NP), jnp.float32),
    )(den.reshape(H * NS, NP))


def _final_body(outp_ref, d0_ref, d1_ref, bias_ref, gw_ref, gb_ref, gms_ref,
                y_ref):
    p0 = outp_ref[:N, :].astype(jnp.float32)
    p1 = outp_ref[NP:NP + N, :].astype(jnp.float32)
    y = 0.5 * (p0 / d0_ref[...] + p1 / d1_ref[...]) + bias_ref[...]
    mu = jnp.mean(y, axis=0, keepdims=True)
    cen = y - gms_ref[...] * mu
    var = jnp.mean(cen * cen, axis=0, keepdims=True)
    y_ref[...] = gw_ref[...] * cen * lax.rsqrt(var + EPS) + gb_ref[...]


def _finish(outp, d0col, d1col, bias, gw, gb, gms):
    return pl.pallas_call(
        _final_body,
        out_shape=jax.ShapeDtypeStruct((N, C), jnp.float32),
    )(outp, d0col, d1col, bias.reshape(1, C), gw.reshape(1, C),
      gb.reshape(1, C), gms.reshape(1, C))


def kernel(x, edge_index, W_src, W_dst, att, bias, gn_weight, gn_bias, gn_mean_scale):
    loops = jnp.arange(N, dtype=jnp.int32)
    pad = jnp.zeros((EP - EPRIME,), jnp.int32)
    srcp = jnp.concatenate([edge_index[0].astype(jnp.int32), loops, pad])
    dstp = jnp.concatenate([edge_index[1].astype(jnp.int32), loops, pad])

    hs2, hd2 = _project(x, W_src, W_dst)
    den, outp = _edge_pass(hs2, hd2, srcp, dstp, att.reshape(H * C))
    denr = _denred(den)
    d0col = denr[0, :N].reshape(N, 1)
    d1col = denr[1, :N].reshape(N, 1)
    return _finish(outp, d0col, d1col, bias, gn_weight, gn_bias,
                   gn_mean_scale)


# speedup vs baseline: 30.0129x; 1.0355x over previous
import functools

import numpy as np

import jax
import jax.numpy as jnp
from jax import lax
from jax.experimental import pallas as pl
from jax.experimental.pallas import tpu as pltpu
from jax.experimental.pallas import tpu_sc as plsc

N = 10000
E = 160000
C = 128
H = 2
SLOPE = 0.2
EPS = 1e-5

NC = 2
NS = 16
L = 16

EPRIME = E + N
K = 48
CPT = 10752
NCH = CPT // K
EP = NS * CPT
NP = 10112
RPT = NP // NS

ROW_BLK = 1000


def _proj_body(x_ref, ws_ref, wd_ref, hs_ref, hd_ref):
    xb = x_ref[...]
    hs_ref[...] = jnp.dot(xb, ws_ref[...], preferred_element_type=jnp.float32)
    hd_ref[...] = jnp.dot(xb, wd_ref[...], preferred_element_type=jnp.float32)


def _project(x, W_src, W_dst):
    nb = N // ROW_BLK
    return pl.pallas_call(
        _proj_body,
        grid=(nb, H),
        in_specs=[
            pl.BlockSpec((ROW_BLK, C), lambda i, j: (i, 0)),
            pl.BlockSpec((C, C), lambda i, j: (0, j)),
            pl.BlockSpec((C, C), lambda i, j: (0, j)),
        ],
        out_specs=[
            pl.BlockSpec((ROW_BLK, C), lambda i, j: (j * nb + i, 0)),
            pl.BlockSpec((ROW_BLK, C), lambda i, j: (j * nb + i, 0)),
        ],
        out_shape=[
            jax.ShapeDtypeStruct((H * N, C), jnp.float32),
            jax.ShapeDtypeStruct((H * N, C), jnp.float32),
        ],
    )(x, W_src, W_dst)


def _edge_body(hs_hbm, hd_hbm, src_hbm, dst_hbm, att_hbm,
               den_hbm, outp_hbm,
               srcv0, dstv0, gidx0, didx0, sbuf0, dbuf0, msgbuf0,
               srcv1, dstv1, gidx1, didx1, sbuf1, dbuf1, msgbuf1,
               tmpa, denp, attq, zbuf, out_spmem,
               sems0, semd0, sems1, semd1, semw0, semw1):
    c = lax.axis_index("c")
    s = lax.axis_index("s")
    cn = c * N
    lane = lax.iota(jnp.int32, L)

    @pl.loop(0, NP // L)
    def _zden(r):
        denp[pl.ds(r * L, L)] = jnp.zeros((L,), jnp.float32)

    @pl.loop(0, 8)
    def _zrow(r):
        for k in range(C // L):
            zbuf[r, pl.ds(k * L, L)] = jnp.zeros((L,), jnp.float32)

    @pl.loop(0, RPT // 8)
    def _zcopy(i):
        pltpu.sync_copy(zbuf, out_spmem.at[pl.ds(s * RPT + i * 8, 8)])

    pltpu.sync_copy(att_hbm.at[pl.ds(c * C, C)], attq)
    plsc.subcore_barrier()

    base = s * CPT

    def prefetch(jj, srcv, dstv, gidx, didx, sbuf, dbuf, sems, semd):
        g0 = base + jj * K
        pltpu.sync_copy(src_hbm.at[pl.ds(g0, K)], srcv)
        pltpu.sync_copy(dst_hbm.at[pl.ds(g0, K)], dstv)

        @pl.loop(0, K // L)
        def _idx(t):
            gidx[pl.ds(t * L, L)] = srcv[pl.ds(t * L, L)] + cn
            didx[pl.ds(t * L, L)] = dstv[pl.ds(t * L, L)] + cn

        pltpu.async_copy(hs_hbm.at[gidx], sbuf, sems)
        pltpu.async_copy(hd_hbm.at[didx], dbuf, semd)

    def compute(jj, dstv, gidx, didx, sbuf, dbuf, msgbuf,
                sems, semd, semw, first):
        pltpu.make_async_copy(hs_hbm.at[gidx], sbuf, sems).wait()
        pltpu.make_async_copy(hd_hbm.at[didx], dbuf, semd).wait()
        if not first:
            @pl.when(jj >= 2)
            def _():
                pltpu.make_async_copy(
                    msgbuf, out_spmem.at[dstv], semw).wait()
        g0c = base + jj * K

        @pl.loop(0, K // L)
        def _grp(g):
            exv = jnp.zeros((L,), jnp.float32)
            gid0 = g0c + g * L
            for e in range(L):
                r = g * L + e
                acc0 = jnp.zeros((L,), jnp.float32)
                acc1 = jnp.zeros((L,), jnp.float32)
                for k in range(0, C // L, 2):
                    sv = sbuf[r, pl.ds(k * L, L)]
                    dv = dbuf[r, pl.ds(k * L, L)]
                    z = sv + dv
                    lr = jnp.maximum(z, z * SLOPE)
                    acc0 = acc0 + lr * attq[pl.ds(k * L, L)]
                    sv1 = sbuf[r, pl.ds((k + 1) * L, L)]
                    dv1 = dbuf[r, pl.ds((k + 1) * L, L)]
                    z1 = sv1 + dv1
                    lr1 = jnp.maximum(z1, z1 * SLOPE)
                    acc1 = acc1 + lr1 * attq[pl.ds((k + 1) * L, L)]
                logit = jnp.full((L,), lax.reduce_sum(acc0 + acc1, (0,)))
                exe = jnp.where(gid0 + e < EPRIME, jnp.exp(logit), 0.0)
                exv = jnp.where(lane == e, exe, exv)
                for k in range(C // L):
                    msgbuf[r, pl.ds(k * L, L)] = sbuf[r, pl.ds(k * L, L)] * exe
            plsc.addupdate_scatter(denp, [dstv[pl.ds(g * L, L)]], exv)

        pltpu.async_copy(msgbuf, out_spmem.at[dstv], semw, add=True)

    prefetch(0, srcv0, dstv0, gidx0, didx0, sbuf0, dbuf0, sems0, semd0)

    @pl.loop(0, NCH, step=2)
    def _chunk(j):
        prefetch(j + 1, srcv1, dstv1, gidx1, didx1, sbuf1, dbuf1,
                 sems1, semd1)
        compute(j, dstv0, gidx0, didx0, sbuf0, dbuf0, msgbuf0,
                sems0, semd0, semw0, first=False)

        @pl.when(j + 2 < NCH)
        def _():
            prefetch(j + 2, srcv0, dstv0, gidx0, didx0, sbuf0, dbuf0,
                     sems0, semd0)

        compute(j + 1, dstv1, gidx1, didx1, sbuf1, dbuf1, msgbuf1,
                sems1, semd1, semw1, first=False)

    pltpu.make_async_copy(msgbuf0, out_spmem.at[dstv0], semw0).wait()
    pltpu.make_async_copy(msgbuf1, out_spmem.at[dstv1], semw1).wait()

    pltpu.sync_copy(denp, den_hbm.at[pl.ds((c * NS + s) * NP, NP)])
    plsc.subcore_barrier()
    pltpu.sync_copy(out_spmem.at[pl.ds(s * RPT, RPT)],
                    outp_hbm.at[pl.ds(c * NP + s * RPT, RPT)])


def _edge_pass(hs2, hd2, srcp, dstp, attf):
    mesh = plsc.VectorSubcoreMesh(core_axis_name="c", subcore_axis_name="s")
    f = functools.partial(
        pl.kernel,
        out_type=[
            jax.ShapeDtypeStruct((H * NS * NP,), jnp.float32),
            jax.ShapeDtypeStruct((H * NP, C), jnp.float32),
        ],
        mesh=mesh,
        compiler_params=pltpu.CompilerParams(needs_layout_passes=False),
        scratch_types=(
            [pltpu.VMEM((K,), jnp.int32)] * 4
            + [pltpu.VMEM((K, C), jnp.float32)] * 3
            + [pltpu.VMEM((K,), jnp.int32)] * 4
            + [pltpu.VMEM((K, C), jnp.float32)] * 3
            + [
                pltpu.VMEM((L,), jnp.float32),
                pltpu.VMEM((NP,), jnp.float32),
                pltpu.VMEM((C,), jnp.float32),
                pltpu.VMEM((8, C), jnp.float32),
                pltpu.VMEM_SHARED((NP, C), jnp.float32),
            ]
            + [pltpu.SemaphoreType.DMA] * 6
        ),
    )(_edge_body)
    return f(hs2, hd2, srcp, dstp, attf)


def _denred_body(den_ref, out_ref):
    out_ref[0:1, :] = jnp.sum(den_ref[:NS, :], axis=0, keepdims=True)
    out_ref[1:2, :] = jnp.sum(den_ref[NS:, :], axis=0, keepdims=True)


def _denred(den):
    return pl.pallas_call(
        _denred_body,
        out_shape=jax.ShapeDtypeStruct((H, NP), jnp.float32),
    )(den.reshape(H * NS, NP))


def _final_body(outp_ref, d0_ref, d1_ref, bias_ref, gw_ref, gb_ref, gms_ref,
                y_ref):
    p0 = outp_ref[:N, :].astype(jnp.float32)
    p1 = outp_ref[NP:NP + N, :].astype(jnp.float32)
    y = 0.5 * (p0 / d0_ref[...] + p1 / d1_ref[...]) + bias_ref[...]
    mu = jnp.mean(y, axis=0, keepdims=True)
    cen = y - gms_ref[...] * mu
    var = jnp.mean(cen * cen, axis=0, keepdims=True)
    y_ref[...] = gw_ref[...] * cen * lax.rsqrt(var + EPS) + gb_ref[...]


def _finish(outp, d0col, d1col, bias, gw, gb, gms):
    return pl.pallas_call(
        _final_body,
        out_shape=jax.ShapeDtypeStruct((N, C), jnp.float32),
    )(outp, d0col, d1col, bias.reshape(1, C), gw.reshape(1, C),
      gb.reshape(1, C), gms.reshape(1, C))


def kernel(x, edge_index, W_src, W_dst, att, bias, gn_weight, gn_bias, gn_mean_scale):
    loops = jnp.arange(N, dtype=jnp.int32)
    pad = jnp.zeros((EP - EPRIME,), jnp.int32)
    srcp = jnp.concatenate([edge_index[0].astype(jnp.int32), loops, pad])
    dstp = jnp.concatenate([edge_index[1].astype(jnp.int32), loops, pad])

    hs2, hd2 = _project(x, W_src, W_dst)
    den, outp = _edge_pass(hs2, hd2, srcp, dstp, att.reshape(H * C))
    denr = _denred(den)
    d0col = denr[0, :N].reshape(N, 1)
    d1col = denr[1, :N].reshape(N, 1)
    return _finish(outp, d0col, d1col, bias, gn_weight, gn_bias,
                   gn_mean_scale)


# speedup vs baseline: 36.7050x; 1.2230x over previous
import functools

import numpy as np

import jax
import jax.numpy as jnp
from jax import lax
from jax.experimental import pallas as pl
from jax.experimental.pallas import tpu as pltpu
from jax.experimental.pallas import tpu_sc as plsc

N = 10000
E = 160000
C = 128
H = 2
SLOPE = 0.2
EPS = 1e-5

NC = 2
NS = 16
L = 16

EPRIME = E + N
K = 48
CPT = 10752
NCH = CPT // K
EP = NS * CPT
NP = 10112
RPT = NP // NS

ROW_BLK = 1000


def _proj_body(x_ref, ws_ref, wd_ref, hs_ref, hd_ref):
    xb = x_ref[...]
    hs_ref[...] = jnp.dot(xb, ws_ref[...], preferred_element_type=jnp.float32)
    hd_ref[...] = jnp.dot(xb, wd_ref[...], preferred_element_type=jnp.float32)


def _project(x, W_src, W_dst):
    nb = N // ROW_BLK
    return pl.pallas_call(
        _proj_body,
        grid=(nb, H),
        in_specs=[
            pl.BlockSpec((ROW_BLK, C), lambda i, j: (i, 0)),
            pl.BlockSpec((C, C), lambda i, j: (0, j)),
            pl.BlockSpec((C, C), lambda i, j: (0, j)),
        ],
        out_specs=[
            pl.BlockSpec((ROW_BLK, C), lambda i, j: (j * nb + i, 0)),
            pl.BlockSpec((ROW_BLK, C), lambda i, j: (j * nb + i, 0)),
        ],
        out_shape=[
            jax.ShapeDtypeStruct((H * N, C), jnp.float32),
            jax.ShapeDtypeStruct((H * N, C), jnp.float32),
        ],
    )(x, W_src, W_dst)


def _edge_body(hs_hbm, hd_hbm, src_hbm, dst_hbm, att_hbm,
               den_hbm, outp_hbm,
               srcv0, dstv0, srcv1, dstv1, srcv2, dstv2, srcv3, dstv3,
               gidx0, didx0, sbuf0, dbuf0, msgbuf0, sidx0,
               gidx1, didx1, sbuf1, dbuf1, msgbuf1, sidx1,
               tmpa, denp, attq, zbuf, out_spmem,
               semi0, semi1, sems0, semd0, sems1, semd1, semw0, semw1):
    c = lax.axis_index("c")
    s = lax.axis_index("s")
    cn = c * N
    lane = lax.iota(jnp.int32, L)

    small = [(srcv0, dstv0), (srcv1, dstv1), (srcv2, dstv2), (srcv3, dstv3)]
    big = [(gidx0, didx0, sbuf0, dbuf0, msgbuf0, sidx0, sems0, semd0, semw0),
           (gidx1, didx1, sbuf1, dbuf1, msgbuf1, sidx1, sems1, semd1, semw1)]
    semi = [semi0, semi1]

    @pl.loop(0, NP // L)
    def _zden(r):
        denp[pl.ds(r * L, L)] = jnp.zeros((L,), jnp.float32)

    @pl.loop(0, 8)
    def _zrow(r):
        for k in range(C // L):
            zbuf[r, pl.ds(k * L, L)] = jnp.zeros((L,), jnp.float32)

    @pl.loop(0, RPT // 8)
    def _zcopy(i):
        pltpu.sync_copy(zbuf, out_spmem.at[pl.ds(s * RPT + i * 8, 8)])

    pltpu.sync_copy(att_hbm.at[pl.ds(c * C, C)], attq)
    plsc.subcore_barrier()

    base = s * CPT

    def idxload(tc, u):
        srcv, dstv = small[u]
        g0 = base + tc * K
        pltpu.async_copy(src_hbm.at[pl.ds(g0, K)], srcv, semi[u % 2])
        pltpu.async_copy(dst_hbm.at[pl.ds(g0, K)], dstv, semi[u % 2])

    def idxwait(tc, u):
        srcv, dstv = small[u]
        g0 = base + tc * K
        pltpu.make_async_copy(src_hbm.at[pl.ds(g0, K)], srcv,
                              semi[u % 2]).wait()
        pltpu.make_async_copy(dst_hbm.at[pl.ds(g0, K)], dstv,
                              semi[u % 2]).wait()

    def gstart(u, p):
        srcv, dstv = small[u]
        gidx, didx, sbuf, dbuf, _, _, sems, semd, _ = big[p]

        @pl.loop(0, K // L)
        def _idx(t):
            gidx[pl.ds(t * L, L)] = srcv[pl.ds(t * L, L)] + cn
            didx[pl.ds(t * L, L)] = dstv[pl.ds(t * L, L)] + cn

        pltpu.async_copy(hs_hbm.at[gidx], sbuf, sems)
        pltpu.async_copy(hd_hbm.at[didx], dbuf, semd)

    def compute(tc, u, p, drain):
        srcv, dstv = small[u]
        gidx, didx, sbuf, dbuf, msgbuf, sidx, sems, semd, semw = big[p]
        pltpu.make_async_copy(hs_hbm.at[gidx], sbuf, sems).wait()
        pltpu.make_async_copy(hd_hbm.at[didx], dbuf, semd).wait()
        if drain:
            @pl.when(tc >= 2)
            def _():
                pltpu.make_async_copy(
                    msgbuf, out_spmem.at[sidx], semw).wait()

        @pl.loop(0, K // L)
        def _sidx(t):
            sidx[pl.ds(t * L, L)] = dstv[pl.ds(t * L, L)]

        g0c = base + tc * K

        @pl.loop(0, K // L)
        def _grp(g):
            exv = jnp.zeros((L,), jnp.float32)
            gid0 = g0c + g * L
            for e in range(L):
                r = g * L + e
                acc0 = jnp.zeros((L,), jnp.float32)
                acc1 = jnp.zeros((L,), jnp.float32)
                for k in range(0, C // L, 2):
                    sv = sbuf[r, pl.ds(k * L, L)]
                    dv = dbuf[r, pl.ds(k * L, L)]
                    z = sv + dv
                    lr = jnp.maximum(z, z * SLOPE)
                    acc0 = acc0 + lr * attq[pl.ds(k * L, L)]
                    sv1 = sbuf[r, pl.ds((k + 1) * L, L)]
                    dv1 = dbuf[r, pl.ds((k + 1) * L, L)]
                    z1 = sv1 + dv1
                    lr1 = jnp.maximum(z1, z1 * SLOPE)
                    acc1 = acc1 + lr1 * attq[pl.ds((k + 1) * L, L)]
                logit = jnp.full((L,), lax.reduce_sum(acc0 + acc1, (0,)))
                exe = jnp.where(gid0 + e < EPRIME, jnp.exp(logit), 0.0)
                exv = jnp.where(lane == e, exe, exv)
                for k in range(C // L):
                    msgbuf[r, pl.ds(k * L, L)] = sbuf[r, pl.ds(k * L, L)] * exe
            plsc.addupdate_scatter(denp, [sidx[pl.ds(g * L, L)]], exv)

        pltpu.async_copy(msgbuf, out_spmem.at[sidx], semw, add=True)

    idxload(0, 0)
    idxload(1, 1)
    idxwait(0, 0)
    gstart(0, 0)

    @pl.loop(0, NCH, step=4)
    def _quad(t):
        for u in range(4):
            tc = t + u

            @pl.when(tc + 2 < NCH)
            def _():
                idxload(tc + 2, (u + 2) % 4)

            @pl.when(tc + 1 < NCH)
            def _():
                idxwait(tc + 1, (u + 1) % 4)
                gstart((u + 1) % 4, (u + 1) % 2)

            compute(tc, u, u % 2, drain=True)

    pltpu.make_async_copy(msgbuf0, out_spmem.at[sidx0], semw0).wait()
    pltpu.make_async_copy(msgbuf1, out_spmem.at[sidx1], semw1).wait()

    pltpu.sync_copy(denp, den_hbm.at[pl.ds((c * NS + s) * NP, NP)])
    plsc.subcore_barrier()
    pltpu.sync_copy(out_spmem.at[pl.ds(s * RPT, RPT)],
                    outp_hbm.at[pl.ds(c * NP + s * RPT, RPT)])


def _edge_pass(hs2, hd2, srcp, dstp, attf):
    mesh = plsc.VectorSubcoreMesh(core_axis_name="c", subcore_axis_name="s")
    f = functools.partial(
        pl.kernel,
        out_type=[
            jax.ShapeDtypeStruct((H * NS * NP,), jnp.float32),
            jax.ShapeDtypeStruct((H * NP, C), jnp.float32),
        ],
        mesh=mesh,
        compiler_params=pltpu.CompilerParams(needs_layout_passes=False),
        scratch_types=(
            [pltpu.VMEM((K,), jnp.int32)] * 8
            + [pltpu.VMEM((K,), jnp.int32)] * 2
            + [pltpu.VMEM((K, C), jnp.float32)] * 3
            + [pltpu.VMEM((K,), jnp.int32)]
            + [pltpu.VMEM((K,), jnp.int32)] * 2
            + [pltpu.VMEM((K, C), jnp.float32)] * 3
            + [pltpu.VMEM((K,), jnp.int32)]
            + [
                pltpu.VMEM((L,), jnp.float32),
                pltpu.VMEM((NP,), jnp.float32),
                pltpu.VMEM((C,), jnp.float32),
                pltpu.VMEM((8, C), jnp.float32),
                pltpu.VMEM_SHARED((NP, C), jnp.float32),
            ]
            + [pltpu.SemaphoreType.DMA] * 8
        ),
    )(_edge_body)
    return f(hs2, hd2, srcp, dstp, attf)


def _denred_body(den_ref, out_ref):
    out_ref[0:1, :] = jnp.sum(den_ref[:NS, :], axis=0, keepdims=True)
    out_ref[1:2, :] = jnp.sum(den_ref[NS:, :], axis=0, keepdims=True)


def _denred(den):
    return pl.pallas_call(
        _denred_body,
        out_shape=jax.ShapeDtypeStruct((H, NP), jnp.float32),
    )(den.reshape(H * NS, NP))


def _final_body(outp_ref, d0_ref, d1_ref, bias_ref, gw_ref, gb_ref, gms_ref,
                y_ref):
    p0 = outp_ref[:N, :].astype(jnp.float32)
    p1 = outp_ref[NP:NP + N, :].astype(jnp.float32)
    y = 0.5 * (p0 / d0_ref[...] + p1 / d1_ref[...]) + bias_ref[...]
    mu = jnp.mean(y, axis=0, keepdims=True)
    cen = y - gms_ref[...] * mu
    var = jnp.mean(cen * cen, axis=0, keepdims=True)
    y_ref[...] = gw_ref[...] * cen * lax.rsqrt(var + EPS) + gb_ref[...]


def _finish(outp, d0col, d1col, bias, gw, gb, gms):
    return pl.pallas_call(
        _final_body,
        out_shape=jax.ShapeDtypeStruct((N, C), jnp.float32),
    )(outp, d0col, d1col, bias.reshape(1, C), gw.reshape(1, C),
      gb.reshape(1, C), gms.reshape(1, C))


def kernel(x, edge_index, W_src, W_dst, att, bias, gn_weight, gn_bias, gn_mean_scale):
    loops = jnp.arange(N, dtype=jnp.int32)
    pad = jnp.zeros((EP - EPRIME,), jnp.int32)
    srcp = jnp.concatenate([edge_index[0].astype(jnp.int32), loops, pad])
    dstp = jnp.concatenate([edge_index[1].astype(jnp.int32), loops, pad])

    hs2, hd2 = _project(x, W_src, W_dst)
    den, outp = _edge_pass(hs2, hd2, srcp, dstp, att.reshape(H * C))
    denr = _denred(den)
    d0col = denr[0, :N].reshape(N, 1)
    d1col = denr[1, :N].reshape(N, 1)
    return _finish(outp, d0col, d1col, bias, gn_weight, gn_bias,
                   gn_mean_scale)
